# baseline (speedup 1.0000x reference)
import functools

import jax
import jax.numpy as jnp
from jax import lax
from jax.experimental import pallas as pl
from jax.experimental.pallas import tpu as pltpu
from jax.experimental.pallas import tpu_sc as plsc

VOCAB = 1000000
EMBED = 64
ROWS = 4096
COLS = 200
B_TOTAL = ROWS * COLS

_info = plsc.get_sparse_core_info()
NC = _info.num_cores
NS = _info.num_subcores
NW = NC * NS
PER_W = B_TOTAL // NW
CH = 128
NCH = PER_W // CH
NBUF = 8
NGRP = NCH // NBUF

_mesh = plsc.VectorSubcoreMesh(core_axis_name="c", subcore_axis_name="s")


@functools.partial(
    pl.kernel,
    mesh=_mesh,
    out_type=jax.ShapeDtypeStruct((NW, NCH, CH, EMBED), jnp.float32),
    scratch_types=[
        pltpu.VMEM((NCH, CH), jnp.int32),
        pltpu.VMEM((NBUF, CH, EMBED), jnp.float32),
        pltpu.SemaphoreType.DMA,
        pltpu.SemaphoreType.DMA,
    ],
    compiler_params=pltpu.CompilerParams(use_tc_tiling_on_sc=False),
)
def _emb_lookup(x_hbm, w_hbm, out_hbm, idx_v, rows_v, gsem, osem):
    wid = lax.axis_index("s") * NC + lax.axis_index("c")
    pltpu.sync_copy(x_hbm.at[wid], idx_v)

    def group(g, carry):
        gathers = []
        for b in range(NBUF):
            j = g * NBUF + b
            gathers.append(
                pltpu.async_copy(w_hbm.at[idx_v.at[j]], rows_v.at[b], gsem)
            )
        for b in range(NBUF):
            gathers[b].wait()
        outs = []
        for b in range(NBUF):
            j = g * NBUF + b
            outs.append(
                pltpu.async_copy(rows_v.at[b], out_hbm.at[wid, j], osem)
            )
        for b in range(NBUF):
            outs[b].wait()
        return carry

    lax.fori_loop(0, NGRP, group, 0)


def kernel(x, w):
    xf = x.reshape(NW, NCH, CH).astype(jnp.int32)
    out = _emb_lookup(xf, w)
    return out.reshape(ROWS, COLS, EMBED)


# speedup vs baseline: 1.0058x; 1.0058x over previous
import functools

import jax
import jax.numpy as jnp
from jax import lax
from jax.experimental import pallas as pl
from jax.experimental.pallas import tpu as pltpu
from jax.experimental.pallas import tpu_sc as plsc

VOCAB = 1000000
EMBED = 64
ROWS = 4096
COLS = 200
B_TOTAL = ROWS * COLS

_info = plsc.get_sparse_core_info()
NC = _info.num_cores
NS = _info.num_subcores
NW = NC * NS
PER_W = B_TOTAL // NW
CH = 128
NCH = PER_W // CH
NBUF = 8
NGRP = NCH // NBUF

_mesh = plsc.VectorSubcoreMesh(core_axis_name="c", subcore_axis_name="s")


@functools.partial(
    pl.kernel,
    mesh=_mesh,
    out_type=jax.ShapeDtypeStruct((NW, NCH, CH, EMBED), jnp.float32),
    scratch_types=[
        pltpu.VMEM((NCH, CH), jnp.int32),
        pltpu.VMEM((NBUF, CH, EMBED), jnp.float32),
        pltpu.SemaphoreType.DMA((NBUF,)),
        pltpu.SemaphoreType.DMA((NBUF,)),
    ],
    compiler_params=pltpu.CompilerParams(use_tc_tiling_on_sc=False),
)
def _emb_lookup(x_hbm, w_hbm, out_hbm, idx_v, rows_v, gsems, osems):
    wid = lax.axis_index("s") * NC + lax.axis_index("c")
    pltpu.sync_copy(x_hbm.at[wid], idx_v)

    def fire_gather(b, j):
        pltpu.async_copy(w_hbm.at[idx_v.at[j]], rows_v.at[b], gsems.at[b])

    def wait_gather(b):
        pltpu.make_async_copy(
            w_hbm.at[pl.ds(0, CH)], rows_v.at[b], gsems.at[b]
        ).wait()

    def fire_out(b, j):
        pltpu.async_copy(rows_v.at[b], out_hbm.at[wid, j], osems.at[b])

    def wait_out(b):
        pltpu.make_async_copy(
            w_hbm.at[pl.ds(0, CH)], out_hbm.at[wid, 0], osems.at[b]
        ).wait()

    for b in range(NBUF):
        fire_gather(b, b)

    def group(g, carry):
        for b in range(NBUF):
            wait_gather(b)
            fire_out(b, g * NBUF + b)
        for b in range(NBUF):
            wait_out(b)
            fire_gather(b, (g + 1) * NBUF + b)
        return carry

    lax.fori_loop(0, NGRP - 1, group, 0)

    gl = NGRP - 1
    for b in range(NBUF):
        wait_gather(b)
        fire_out(b, gl * NBUF + b)
    for b in range(NBUF):
        wait_out(b)


def kernel(x, w):
    xf = x.reshape(NW, NCH, CH).astype(jnp.int32)
    out = _emb_lookup(xf, w)
    return out.reshape(ROWS, COLS, EMBED)


# speedup vs baseline: 1.0058x; 1.0000x over previous
import functools

import jax
import jax.numpy as jnp
from jax import lax
from jax.experimental import pallas as pl
from jax.experimental.pallas import tpu as pltpu
from jax.experimental.pallas import tpu_sc as plsc

VOCAB = 1000000
EMBED = 64
ROWS = 4096
COLS = 200
B_TOTAL = ROWS * COLS

_info = plsc.get_sparse_core_info()
NC = _info.num_cores
NS = _info.num_subcores
NW = NC * NS
PER_W = B_TOTAL // NW
CH = 128
NCH = PER_W // CH
NBUF = 8
NGRP = NCH // NBUF

_mesh = plsc.VectorSubcoreMesh(core_axis_name="c", subcore_axis_name="s")


@functools.partial(
    pl.kernel,
    mesh=_mesh,
    out_type=jax.ShapeDtypeStruct((NW, NCH, CH, EMBED), jnp.float32),
    scratch_types=[
        pltpu.VMEM((NCH, CH), jnp.int32),
        pltpu.VMEM((NBUF, CH, EMBED), jnp.float32),
        pltpu.SemaphoreType.DMA((NBUF,)),
        pltpu.SemaphoreType.DMA((NBUF,)),
    ],
    compiler_params=pltpu.CompilerParams(use_tc_tiling_on_sc=False),
)
def _emb_lookup(x_hbm, w_hbm, out_hbm, idx_v, rows_v, gsems, osems):
    wid = lax.axis_index("s") * NC + lax.axis_index("c")
    pltpu.sync_copy(x_hbm.at[wid], idx_v)

    def fire_gather(b, j):
        pltpu.async_copy(w_hbm.at[idx_v.at[j]], rows_v.at[b], gsems.at[b])

    def wait_gather(b):
        pltpu.make_async_copy(
            w_hbm.at[idx_v.at[0]], rows_v.at[b], gsems.at[b]
        ).wait()

    def fire_out(b, j):
        pltpu.async_copy(rows_v.at[b], out_hbm.at[wid, j], osems.at[b])

    def wait_out(b):
        pltpu.make_async_copy(
            rows_v.at[b], out_hbm.at[wid, 0], osems.at[b]
        ).wait()

    for b in range(NBUF):
        fire_gather(b, b)

    def group(g, carry):
        for b in range(NBUF):
            wait_gather(b)
            fire_out(b, g * NBUF + b)
        for b in range(NBUF):
            wait_out(b)
            fire_gather(b, (g + 1) * NBUF + b)
        return carry

    lax.fori_loop(0, NGRP - 1, group, 0)

    gl = NGRP - 1
    for b in range(NBUF):
        wait_gather(b)
        fire_out(b, gl * NBUF + b)
    for b in range(NBUF):
        wait_out(b)


def kernel(x, w):
    xf = x.reshape(NW, NCH, CH).astype(jnp.int32)
    out = _emb_lookup(xf, w)
    return out.reshape(ROWS, COLS, EMBED)


# speedup vs baseline: 1.0066x; 1.0008x over previous
import functools

import jax
import jax.numpy as jnp
from jax import lax
from jax.experimental import pallas as pl
from jax.experimental.pallas import tpu as pltpu
from jax.experimental.pallas import tpu_sc as plsc

VOCAB = 1000000
EMBED = 64
ROWS = 4096
COLS = 200

_info = plsc.get_sparse_core_info()
NC = _info.num_cores
NS = _info.num_subcores
NW = NC * NS
ROWS_W = ROWS // NW
CHA = 128
CHB = COLS - CHA
NBUF = 8
T = ROWS_W * 2
NGRP = T // NBUF

_mesh = plsc.VectorSubcoreMesh(core_axis_name="c", subcore_axis_name="s")


@functools.partial(
    pl.kernel,
    mesh=_mesh,
    out_type=jax.ShapeDtypeStruct((ROWS, COLS, EMBED), jnp.float32),
    scratch_types=[
        pltpu.VMEM((ROWS_W, COLS), jnp.int32),
        pltpu.VMEM((NBUF // 2, CHA, EMBED), jnp.float32),
        pltpu.VMEM((NBUF // 2, CHB, EMBED), jnp.float32),
        pltpu.SemaphoreType.DMA((NBUF,)),
        pltpu.SemaphoreType.DMA((NBUF,)),
    ],
    compiler_params=pltpu.CompilerParams(use_tc_tiling_on_sc=False),
)
def _emb_lookup(x_hbm, w_hbm, out_hbm, idx_v, rows_a, rows_b, gsems, osems):
    wid = lax.axis_index("s") * NC + lax.axis_index("c")
    r0 = wid * ROWS_W
    pltpu.sync_copy(x_hbm.at[pl.ds(r0, ROWS_W)], idx_v)

    def buf(b):
        if b % 2 == 0:
            return rows_a.at[b // 2], 0, CHA
        return rows_b.at[b // 2], CHA, CHB

    def fire_gather(b, t):
        row = t // 2
        dst, col, ch = buf(b)
        pltpu.async_copy(
            w_hbm.at[idx_v.at[row, pl.ds(col, ch)]], dst, gsems.at[b]
        )

    def wait_gather(b):
        dst, col, ch = buf(b)
        pltpu.make_async_copy(
            w_hbm.at[idx_v.at[0, pl.ds(col, ch)]], dst, gsems.at[b]
        ).wait()

    def fire_out(b, t):
        row = t // 2
        src, col, ch = buf(b)
        pltpu.async_copy(
            src, out_hbm.at[r0 + row, pl.ds(col, ch)], osems.at[b]
        )

    def wait_out(b):
        src, col, ch = buf(b)
        pltpu.make_async_copy(
            src, out_hbm.at[r0, pl.ds(col, ch)], osems.at[b]
        ).wait()

    for b in range(NBUF):
        fire_gather(b, b)

    def group(g, carry):
        for b in range(NBUF):
            wait_gather(b)
            fire_out(b, g * NBUF + b)
        for b in range(NBUF):
            wait_out(b)
            fire_gather(b, (g + 1) * NBUF + b)
        return carry

    lax.fori_loop(0, NGRP - 1, group, 0)

    gl = NGRP - 1
    for b in range(NBUF):
        wait_gather(b)
        fire_out(b, gl * NBUF + b)
    for b in range(NBUF):
        wait_out(b)


def kernel(x, w):
    return _emb_lookup(x, w)
